# baseline (device time: 169973 ns/iter reference)
import jax
import jax.numpy as jnp
from jax import lax
from jax.experimental import pallas as pl
from jax.experimental.pallas import tpu as pltpu

N_DEV = 4


def kernel(x, w_mat, scale_x, scale_w):
    m_per, k = x.shape
    _, n_per = w_mat.shape

    def body(x_ref, w_ref, sx_ref, sw_ref, out_ref, comm_ref,
             send_sems, recv_sems):
        my_pos = lax.axis_index("i")
        left = (my_pos + N_DEV - 1) % N_DEV
        right = (my_pos + 1) % N_DEV

        barrier_sem = pltpu.get_barrier_semaphore()
        for nbr in (left, right):
            pl.semaphore_signal(
                barrier_sem, inc=1,
                device_id=(nbr,), device_id_type=pl.DeviceIdType.MESH,
            )
        pl.semaphore_wait(barrier_sem, 2)

        scale = sx_ref[0] * sw_ref[0]

        def chunk_gemm(origin, chunk):
            acc = jnp.dot(chunk, w_ref[:, :], preferred_element_type=jnp.int32)
            y = acc.astype(jnp.float32) * scale
            out_ref[pl.ds(origin * m_per, m_per), :] = jnp.maximum(y, 0.0)

        comm_ref[0, :, :] = x_ref[:, :]
        chunk_gemm(my_pos, x_ref[:, :])

        for h in range(N_DEV - 1):
            send_slot = h % 2
            recv_slot = (h + 1) % 2
            rdma = pltpu.make_async_remote_copy(
                src_ref=comm_ref.at[send_slot],
                dst_ref=comm_ref.at[recv_slot],
                send_sem=send_sems.at[send_slot],
                recv_sem=recv_sems.at[recv_slot],
                device_id=(right,),
                device_id_type=pl.DeviceIdType.MESH,
            )
            rdma.start()
            rdma.wait()

            origin = (my_pos + N_DEV - h - 1) % N_DEV
            chunk_gemm(origin, comm_ref[recv_slot, :, :])

    return pl.pallas_call(
        body,
        out_shape=jax.ShapeDtypeStruct((N_DEV * m_per, n_per), jnp.float32),
        in_specs=[
            pl.BlockSpec(memory_space=pltpu.VMEM),
            pl.BlockSpec(memory_space=pltpu.VMEM),
            pl.BlockSpec(memory_space=pltpu.SMEM),
            pl.BlockSpec(memory_space=pltpu.SMEM),
        ],
        out_specs=pl.BlockSpec(memory_space=pltpu.VMEM),
        scratch_shapes=[
            pltpu.VMEM((2, m_per, k), jnp.int8),
            pltpu.SemaphoreType.DMA((2,)),
            pltpu.SemaphoreType.DMA((2,)),
        ],
        compiler_params=pltpu.CompilerParams(collective_id=0),
    )(x, w_mat, scale_x, scale_w)


# device time: 90447 ns/iter; 1.8793x vs baseline; 1.8793x over previous
import jax
import jax.numpy as jnp
from jax import lax
from jax.experimental import pallas as pl
from jax.experimental.pallas import tpu as pltpu

N_DEV = 4


def kernel(x, w_mat, scale_x, scale_w):
    m_per, k = x.shape
    _, n_per = w_mat.shape
    m_half = m_per // 2

    def body(x_ref, w_ref, sx_ref, sw_ref, out_ref,
             buf_l, buf_r, buf_d, send_sems, recv_sems):
        my_pos = lax.axis_index("i")
        left = (my_pos + N_DEV - 1) % N_DEV
        right = (my_pos + 1) % N_DEV

        barrier_sem = pltpu.get_barrier_semaphore()
        for nbr in (left, right):
            pl.semaphore_signal(
                barrier_sem, inc=1,
                device_id=(nbr,), device_id_type=pl.DeviceIdType.MESH,
            )
        pl.semaphore_wait(barrier_sem, 2)

        send_r = pltpu.make_async_remote_copy(
            src_ref=x_ref, dst_ref=buf_l,
            send_sem=send_sems.at[0], recv_sem=recv_sems.at[0],
            device_id=(right,), device_id_type=pl.DeviceIdType.MESH,
        )
        send_l = pltpu.make_async_remote_copy(
            src_ref=x_ref, dst_ref=buf_r,
            send_sem=send_sems.at[1], recv_sem=recv_sems.at[1],
            device_id=(left,), device_id_type=pl.DeviceIdType.MESH,
        )
        send_r.start()
        send_l.start()

        scale = sx_ref[0] * sw_ref[0]

        def chunk_gemm(origin, chunk):
            acc = jnp.dot(chunk, w_ref[:, :], preferred_element_type=jnp.int32)
            y = acc.astype(jnp.float32) * scale
            out_ref[pl.ds(origin * m_per, m_per), :] = jnp.maximum(y, 0.0)

        chunk_gemm(my_pos, x_ref[:, :])

        send_r.wait_recv()
        fwd_r = pltpu.make_async_remote_copy(
            src_ref=buf_l.at[pl.ds(0, m_half)],
            dst_ref=buf_d.at[pl.ds(0, m_half)],
            send_sem=send_sems.at[2], recv_sem=recv_sems.at[2],
            device_id=(right,), device_id_type=pl.DeviceIdType.MESH,
        )
        fwd_r.start()
        chunk_gemm(left, buf_l[:, :])

        send_l.wait_recv()
        fwd_l = pltpu.make_async_remote_copy(
            src_ref=buf_r.at[pl.ds(m_half, m_half)],
            dst_ref=buf_d.at[pl.ds(m_half, m_half)],
            send_sem=send_sems.at[3], recv_sem=recv_sems.at[3],
            device_id=(left,), device_id_type=pl.DeviceIdType.MESH,
        )
        fwd_l.start()
        chunk_gemm(right, buf_r[:, :])

        fwd_r.wait_recv()
        fwd_l.wait_recv()
        diag = (my_pos + 2) % N_DEV
        chunk_gemm(diag, buf_d[:, :])

        send_r.wait_send()
        send_l.wait_send()
        fwd_r.wait_send()
        fwd_l.wait_send()

    return pl.pallas_call(
        body,
        out_shape=jax.ShapeDtypeStruct((N_DEV * m_per, n_per), jnp.float32),
        in_specs=[
            pl.BlockSpec(memory_space=pltpu.VMEM),
            pl.BlockSpec(memory_space=pltpu.VMEM),
            pl.BlockSpec(memory_space=pltpu.SMEM),
            pl.BlockSpec(memory_space=pltpu.SMEM),
        ],
        out_specs=pl.BlockSpec(memory_space=pltpu.VMEM),
        scratch_shapes=[
            pltpu.VMEM((m_per, k), jnp.int8),
            pltpu.VMEM((m_per, k), jnp.int8),
            pltpu.VMEM((m_per, k), jnp.int8),
            pltpu.SemaphoreType.DMA((4,)),
            pltpu.SemaphoreType.DMA((4,)),
        ],
        compiler_params=pltpu.CompilerParams(collective_id=0),
    )(x, w_mat, scale_x, scale_w)


# device time: 85773 ns/iter; 1.9817x vs baseline; 1.0545x over previous
import jax
import jax.numpy as jnp
from jax import lax
from jax.experimental import pallas as pl
from jax.experimental.pallas import tpu as pltpu

N_DEV = 4


def kernel(x, w_mat, scale_x, scale_w):
    m_per, k = x.shape
    _, n_per = w_mat.shape
    m_q = m_per // 4

    def body(x_hbm, w_hbm, sx_ref, sw_ref, out_hbm,
             x_vmem, w_vmem, buf_l, buf_r, buf_d, stage,
             in_sems, send_sems, recv_sems,
             fwd_send_sems, fwd_recv_sems, out_sems):
        my_pos = lax.axis_index("i")
        left = (my_pos + N_DEV - 1) % N_DEV
        right = (my_pos + 1) % N_DEV

        cp_x = pltpu.make_async_copy(x_hbm, x_vmem, in_sems.at[0])
        cp_w = pltpu.make_async_copy(w_hbm, w_vmem, in_sems.at[1])
        cp_x.start()
        cp_w.start()

        barrier_sem = pltpu.get_barrier_semaphore()
        for nbr in (left, right):
            pl.semaphore_signal(
                barrier_sem, inc=1,
                device_id=(nbr,), device_id_type=pl.DeviceIdType.MESH,
            )
        pl.semaphore_wait(barrier_sem, 2)

        cp_x.wait()
        send_r = pltpu.make_async_remote_copy(
            src_ref=x_vmem, dst_ref=buf_l,
            send_sem=send_sems.at[0], recv_sem=recv_sems.at[0],
            device_id=(right,), device_id_type=pl.DeviceIdType.MESH,
        )
        send_l = pltpu.make_async_remote_copy(
            src_ref=x_vmem, dst_ref=buf_r,
            send_sem=send_sems.at[1], recv_sem=recv_sems.at[1],
            device_id=(left,), device_id_type=pl.DeviceIdType.MESH,
        )
        send_r.start()
        send_l.start()

        cp_w.wait()
        scale = sx_ref[0] * sw_ref[0]

        def chunk_gemm(slot, rows, origin_row, chunk, sem_idx):
            acc = jnp.dot(chunk, w_vmem[:, :],
                          preferred_element_type=jnp.int32)
            stage[slot, pl.ds(rows, chunk.shape[0]), :] = jnp.maximum(
                acc.astype(jnp.float32) * scale, 0.0)
            cp = pltpu.make_async_copy(
                stage.at[slot, pl.ds(rows, chunk.shape[0])],
                out_hbm.at[pl.ds(origin_row, chunk.shape[0])],
                out_sems.at[sem_idx],
            )
            cp.start()
            return cp

        cp0 = chunk_gemm(0, 0, my_pos * m_per, x_vmem[:, :], 0)

        def fwd_strip(src_buf, j, target, sem_idx):
            f = pltpu.make_async_remote_copy(
                src_ref=src_buf.at[pl.ds(j * m_q, m_q)],
                dst_ref=buf_d.at[pl.ds(j * m_q, m_q)],
                send_sem=fwd_send_sems.at[sem_idx],
                recv_sem=fwd_recv_sems.at[sem_idx],
                device_id=(target,), device_id_type=pl.DeviceIdType.MESH,
            )
            f.start()
            return f

        send_r.wait_recv()
        f0 = fwd_strip(buf_l, 0, right, 0)
        f1 = fwd_strip(buf_l, 1, right, 1)

        send_l.wait_recv()
        f2 = fwd_strip(buf_r, 2, left, 2)
        f3 = fwd_strip(buf_r, 3, left, 3)

        cp1 = chunk_gemm(1, 0, left * m_per, buf_l[:, :], 1)
        cp2 = chunk_gemm(2, 0, right * m_per, buf_r[:, :], 2)

        diag = (my_pos + 2) % N_DEV
        diag_cps = []
        for j in (0, 2, 1, 3):
            (f0, f1, f2, f3)[j].wait_recv()
            diag_cps.append(chunk_gemm(
                3, j * m_q, diag * m_per + j * m_q,
                buf_d[pl.ds(j * m_q, m_q), :], 3 + j))

        send_r.wait_send()
        send_l.wait_send()
        for f in (f0, f1, f2, f3):
            f.wait_send()
        for cp in (cp0, cp1, cp2, *diag_cps):
            cp.wait()

    x = pltpu.with_memory_space_constraint(x, pltpu.MemorySpace.HBM)
    w_mat = pltpu.with_memory_space_constraint(w_mat, pltpu.MemorySpace.HBM)
    return pl.pallas_call(
        body,
        out_shape=pltpu.MemorySpace.HBM((N_DEV * m_per, n_per), jnp.float32),
        in_specs=[
            pl.BlockSpec(memory_space=pltpu.MemorySpace.HBM),
            pl.BlockSpec(memory_space=pltpu.MemorySpace.HBM),
            pl.BlockSpec(memory_space=pltpu.SMEM),
            pl.BlockSpec(memory_space=pltpu.SMEM),
        ],
        out_specs=pl.BlockSpec(memory_space=pltpu.MemorySpace.HBM),
        scratch_shapes=[
            pltpu.VMEM((m_per, k), jnp.int8),
            pltpu.VMEM((k, n_per), jnp.int8),
            pltpu.VMEM((m_per, k), jnp.int8),
            pltpu.VMEM((m_per, k), jnp.int8),
            pltpu.VMEM((m_per, k), jnp.int8),
            pltpu.VMEM((4, m_per, n_per), jnp.float32),
            pltpu.SemaphoreType.DMA((2,)),
            pltpu.SemaphoreType.DMA((2,)),
            pltpu.SemaphoreType.DMA((2,)),
            pltpu.SemaphoreType.DMA((4,)),
            pltpu.SemaphoreType.DMA((4,)),
            pltpu.SemaphoreType.DMA((7,)),
        ],
        compiler_params=pltpu.CompilerParams(collective_id=0),
    )(x, w_mat, scale_x, scale_w)
